# baseline (speedup 1.0000x reference)
import functools

import jax
import jax.numpy as jnp
from jax import lax
from jax.experimental import pallas as pl
from jax.experimental.pallas import tpu as pltpu
from jax.experimental.pallas import tpu_sc as plsc

_BATCH = 16384
_DIM = 64
_NC = 2
_NS = 16
_NW = _NC * _NS
_BPW = _BATCH // _NW
_CHUNK = 128
_HALF = _BPW // 2


def _sc_gather(uidx2d, iidx2d, user_packed, item_packed):
    mesh = plsc.VectorSubcoreMesh(core_axis_name="c", subcore_axis_name="s")

    @functools.partial(
        pl.kernel,
        out_type=[
            jax.ShapeDtypeStruct((_BATCH, 128), jnp.float32),
            jax.ShapeDtypeStruct((_BATCH, 128), jnp.float32),
        ],
        mesh=mesh,
        scratch_types=[
            pltpu.VMEM((_BPW // _CHUNK, _CHUNK), jnp.int32),
            pltpu.VMEM((_BPW // _CHUNK, _CHUNK), jnp.int32),
            pltpu.VMEM((_HALF, 128), jnp.float32),
            pltpu.VMEM((_HALF, 128), jnp.float32),
            pltpu.SemaphoreType.DMA,
        ],
    )
    def k(user_hbm, item_hbm, uidx_hbm, iidx_hbm, uout_hbm, iout_hbm,
          uidx_v, iidx_v, urows_v, irows_v, sem):
        wid = lax.axis_index("s") * _NC + lax.axis_index("c")
        base = wid * _BPW
        nidx = _BPW // _CHUNK
        pltpu.sync_copy(uidx_hbm.at[pl.ds(wid * nidx, nidx)], uidx_v)
        pltpu.sync_copy(iidx_hbm.at[pl.ds(wid * nidx, nidx)], iidx_v)
        chunks_per_half = _HALF // _CHUNK
        for h in range(2):
            copies = []
            for j in range(chunks_per_half):
                jj = h * chunks_per_half + j
                copies.append(pltpu.async_copy(
                    user_hbm.at[uidx_v.at[jj]],
                    urows_v.at[pl.ds(j * _CHUNK, _CHUNK)], sem))
                copies.append(pltpu.async_copy(
                    item_hbm.at[iidx_v.at[jj]],
                    irows_v.at[pl.ds(j * _CHUNK, _CHUNK)], sem))
            for c in copies:
                c.wait()
            pltpu.sync_copy(urows_v, uout_hbm.at[pl.ds(base + h * _HALF, _HALF)])
            pltpu.sync_copy(irows_v, iout_hbm.at[pl.ds(base + h * _HALF, _HALF)])

    return k(user_packed, item_packed, uidx2d, iidx2d)


_PACK_W = 4096


_HALF_W = _PACK_W // 2


def _pack_body(xt_ref, o_ref):
    x = xt_ref[...]
    row = jax.lax.broadcasted_iota(jnp.int32, (_DIM, _DIM), 0)
    col = jax.lax.broadcasted_iota(jnp.int32, (_DIM, _DIM), 1)
    ident = (row == col).astype(jnp.float32)
    dn = (((0,), (0,)), ((), ()))
    o_ref[:, 0:_DIM] = jax.lax.dot_general(
        x[:, 0:_HALF_W], ident, dn, preferred_element_type=jnp.float32)
    o_ref[:, _DIM:128] = jax.lax.dot_general(
        x[:, _HALF_W:_PACK_W], ident, dn, preferred_element_type=jnp.float32)


def _pack(table_t):
    n = table_t.shape[1]
    grid = (n + _PACK_W - 1) // _PACK_W
    return pl.pallas_call(
        _pack_body,
        grid=(grid,),
        in_specs=[pl.BlockSpec((_DIM, _PACK_W), lambda g: (0, g))],
        out_specs=pl.BlockSpec((_HALF_W, 128), lambda g: (g, 0)),
        out_shape=jax.ShapeDtypeStruct((grid * _HALF_W, 128), jnp.float32),
    )(table_t)


_TC_BLOCK = 2048


def _tc_body(u_ref, i_ref, pu_ref, pi_ref, p_ref, o_ref):
    w = p_ref[0:1, 0:_DIM]
    b = p_ref[1, 0]
    pu = pu_ref[...] == 1
    pi = pi_ref[...] == 1
    u = jnp.where(pu, u_ref[:, _DIM:2 * _DIM], u_ref[:, 0:_DIM])
    v = jnp.where(pi, i_ref[:, _DIM:2 * _DIM], i_ref[:, 0:_DIM])
    logits = jnp.sum(u * v * w, axis=1, keepdims=True) + b
    o_ref[...] = jax.nn.sigmoid(logits)


def _tc_tail(gu, gi, pu, pi, params):
    grid = _BATCH // _TC_BLOCK
    return pl.pallas_call(
        _tc_body,
        grid=(grid,),
        in_specs=[
            pl.BlockSpec((_TC_BLOCK, 128), lambda g: (g, 0)),
            pl.BlockSpec((_TC_BLOCK, 128), lambda g: (g, 0)),
            pl.BlockSpec((_TC_BLOCK, 1), lambda g: (g, 0)),
            pl.BlockSpec((_TC_BLOCK, 1), lambda g: (g, 0)),
            pl.BlockSpec((8, 128), lambda g: (0, 0)),
        ],
        out_specs=pl.BlockSpec((_TC_BLOCK, 1), lambda g: (g, 0)),
        out_shape=jax.ShapeDtypeStruct((_BATCH, 1), jnp.float32),
    )(gu, gi, pu, pi, params)


def kernel(user_indices, item_indices, embedding_user, embedding_item,
           affine_w, affine_b):
    uidx = user_indices.astype(jnp.int32)
    iidx = item_indices.astype(jnp.int32)
    user_packed = _pack(embedding_user.T)
    item_packed = _pack(embedding_item.T)
    urow = ((uidx >> 12) << 11) | (uidx & 2047)
    irow = ((iidx >> 12) << 11) | (iidx & 2047)
    uidx2d = urow.reshape(_BATCH // _CHUNK, _CHUNK)
    iidx2d = irow.reshape(_BATCH // _CHUNK, _CHUNK)
    pu = ((uidx >> 11) & 1).reshape(_BATCH, 1)
    pi = ((iidx >> 11) & 1).reshape(_BATCH, 1)
    gu, gi = _sc_gather(uidx2d, iidx2d, user_packed, item_packed)
    params = jnp.zeros((8, 128), jnp.float32)
    params = params.at[0, 0:_DIM].set(affine_w.reshape(_DIM))
    params = params.at[1, 0].set(affine_b[0])
    return _tc_tail(gu, gi, pu, pi, params)


# speedup vs baseline: 1.3413x; 1.3413x over previous
import functools

import jax
import jax.numpy as jnp
from jax import lax
from jax.experimental import pallas as pl
from jax.experimental.pallas import tpu as pltpu
from jax.experimental.pallas import tpu_sc as plsc

_BATCH = 16384
_DIM = 64
_NC = 2
_NS = 16
_NW = _NC * _NS
_BPW = _BATCH // _NW
_CHUNK = 128
_HALF = _BPW // 2


def _sc_gather(uidx2d, iidx2d, user_packed, item_packed):
    mesh = plsc.VectorSubcoreMesh(core_axis_name="c", subcore_axis_name="s")

    @functools.partial(
        pl.kernel,
        out_type=[
            jax.ShapeDtypeStruct((_BATCH, 128), jnp.float32),
            jax.ShapeDtypeStruct((_BATCH, 128), jnp.float32),
        ],
        mesh=mesh,
        scratch_types=[
            pltpu.VMEM((_BPW // _CHUNK, _CHUNK), jnp.int32),
            pltpu.VMEM((_BPW // _CHUNK, _CHUNK), jnp.int32),
            pltpu.VMEM((_HALF, 128), jnp.float32),
            pltpu.VMEM((_HALF, 128), jnp.float32),
            pltpu.SemaphoreType.DMA,
        ],
    )
    def k(user_hbm, item_hbm, uidx_hbm, iidx_hbm, uout_hbm, iout_hbm,
          uidx_v, iidx_v, urows_v, irows_v, sem):
        wid = lax.axis_index("s") * _NC + lax.axis_index("c")
        base = wid * _BPW
        nidx = _BPW // _CHUNK
        pltpu.sync_copy(uidx_hbm.at[pl.ds(wid * nidx, nidx)], uidx_v)
        pltpu.sync_copy(iidx_hbm.at[pl.ds(wid * nidx, nidx)], iidx_v)
        chunks_per_half = _HALF // _CHUNK
        for h in range(2):
            copies = []
            for j in range(chunks_per_half):
                jj = h * chunks_per_half + j
                copies.append(pltpu.async_copy(
                    user_hbm.at[uidx_v.at[jj]],
                    urows_v.at[pl.ds(j * _CHUNK, _CHUNK)], sem))
                copies.append(pltpu.async_copy(
                    item_hbm.at[iidx_v.at[jj]],
                    irows_v.at[pl.ds(j * _CHUNK, _CHUNK)], sem))
            for c in copies:
                c.wait()
            pltpu.sync_copy(urows_v, uout_hbm.at[pl.ds(base + h * _HALF, _HALF)])
            pltpu.sync_copy(irows_v, iout_hbm.at[pl.ds(base + h * _HALF, _HALF)])

    return k(user_packed, item_packed, uidx2d, iidx2d)


_PACK_W = 8192


_HALF_W = _PACK_W // 2


def _pack_body(xt_ref, o_ref):
    x = xt_ref[...].astype(jnp.bfloat16)
    row = jax.lax.broadcasted_iota(jnp.int32, (_DIM, _DIM), 0)
    col = jax.lax.broadcasted_iota(jnp.int32, (_DIM, _DIM), 1)
    ident = (row == col).astype(jnp.bfloat16)
    dn = (((0,), (0,)), ((), ()))
    o_ref[:, 0:_DIM] = jax.lax.dot_general(
        x[:, 0:_HALF_W], ident, dn, preferred_element_type=jnp.float32)
    o_ref[:, _DIM:128] = jax.lax.dot_general(
        x[:, _HALF_W:_PACK_W], ident, dn, preferred_element_type=jnp.float32)


def _pack(table_t):
    n = table_t.shape[1]
    grid = (n + _PACK_W - 1) // _PACK_W
    return pl.pallas_call(
        _pack_body,
        grid=(grid,),
        in_specs=[pl.BlockSpec((_DIM, _PACK_W), lambda g: (0, g))],
        out_specs=pl.BlockSpec((_HALF_W, 128), lambda g: (g, 0)),
        out_shape=jax.ShapeDtypeStruct((grid * _HALF_W, 128), jnp.float32),
    )(table_t)


_TC_BLOCK = 2048


def _tc_body(u_ref, i_ref, pu_ref, pi_ref, p_ref, o_ref):
    w = p_ref[0:1, 0:_DIM]
    b = p_ref[1, 0]
    pu = pu_ref[...] == 1
    pi = pi_ref[...] == 1
    u = jnp.where(pu, u_ref[:, _DIM:2 * _DIM], u_ref[:, 0:_DIM])
    v = jnp.where(pi, i_ref[:, _DIM:2 * _DIM], i_ref[:, 0:_DIM])
    logits = jnp.sum(u * v * w, axis=1, keepdims=True) + b
    o_ref[...] = jax.nn.sigmoid(logits)


def _tc_tail(gu, gi, pu, pi, params):
    grid = _BATCH // _TC_BLOCK
    return pl.pallas_call(
        _tc_body,
        grid=(grid,),
        in_specs=[
            pl.BlockSpec((_TC_BLOCK, 128), lambda g: (g, 0)),
            pl.BlockSpec((_TC_BLOCK, 128), lambda g: (g, 0)),
            pl.BlockSpec((_TC_BLOCK, 1), lambda g: (g, 0)),
            pl.BlockSpec((_TC_BLOCK, 1), lambda g: (g, 0)),
            pl.BlockSpec((8, 128), lambda g: (0, 0)),
        ],
        out_specs=pl.BlockSpec((_TC_BLOCK, 1), lambda g: (g, 0)),
        out_shape=jax.ShapeDtypeStruct((_BATCH, 1), jnp.float32),
    )(gu, gi, pu, pi, params)


def kernel(user_indices, item_indices, embedding_user, embedding_item,
           affine_w, affine_b):
    uidx = user_indices.astype(jnp.int32)
    iidx = item_indices.astype(jnp.int32)
    user_packed = _pack(embedding_user.T)
    item_packed = _pack(embedding_item.T)
    urow = (uidx // _PACK_W) * _HALF_W + (uidx % _HALF_W)
    irow = (iidx // _PACK_W) * _HALF_W + (iidx % _HALF_W)
    uidx2d = urow.reshape(_BATCH // _CHUNK, _CHUNK)
    iidx2d = irow.reshape(_BATCH // _CHUNK, _CHUNK)
    pu = ((uidx // _HALF_W) % 2).reshape(_BATCH, 1)
    pi = ((iidx // _HALF_W) % 2).reshape(_BATCH, 1)
    gu, gi = _sc_gather(uidx2d, iidx2d, user_packed, item_packed)
    params = jnp.zeros((8, 128), jnp.float32)
    params = params.at[0, 0:_DIM].set(affine_w.reshape(_DIM))
    params = params.at[1, 0].set(affine_b[0])
    return _tc_tail(gu, gi, pu, pi, params)
